# baseline (speedup 1.0000x reference)
import functools

import jax
import jax.numpy as jnp
from jax import lax
from jax.experimental import pallas as pl
from jax.experimental.pallas import tpu as pltpu
from jax.experimental.pallas import tpu_sc as plsc

N = 10000
F = 256
BW = 80
NBAND = 125
NBPAD = 128
SPLIT = 5200
NW = 32
RW = (N - SPLIT) // NW
BCAP = 128
ECAP = 192
NB_FIX = 40
NE_FIX = 56
NGB_UNC = 5
NGE_UNC = 7


def _mm_kernel(x_ref, w_ref, o_ref):
    o_ref[...] = jnp.dot(x_ref[...], w_ref[...],
                         preferred_element_type=jnp.float32).astype(jnp.bfloat16)


def _feat_mm(x, w):
    br = 1000
    return pl.pallas_call(
        _mm_kernel,
        grid=(N // br,),
        in_specs=[
            pl.BlockSpec((br, F), lambda i: (i, 0)),
            pl.BlockSpec((F, F), lambda i: (0, 0)),
        ],
        out_specs=pl.BlockSpec((br, F), lambda i: (i, 0)),
        out_shape=jax.ShapeDtypeStruct((N, F), jnp.bfloat16),
    )(x, w)


def _l1_kernel(adj_ref, y_ref, b_ref, bd_ref, h_ref, bs_ref):
    a16 = adj_ref[...].astype(jnp.bfloat16)
    acc = jnp.dot(a16, y_ref[...], preferred_element_type=jnp.float32)
    h_ref[...] = jnp.maximum(acc + b_ref[...], 0.0).astype(jnp.bfloat16)
    bs_ref[...] = jnp.dot(a16, bd_ref[...],
                          preferred_element_type=jnp.float32)


def _layer1(adj, y1, b1, bdiag):
    br = 200
    return pl.pallas_call(
        _l1_kernel,
        grid=(N // br,),
        in_specs=[
            pl.BlockSpec((br, N), lambda i: (i, 0)),
            pl.BlockSpec((N, F), lambda i: (0, 0)),
            pl.BlockSpec((1, F), lambda i: (0, 0)),
            pl.BlockSpec((N, NBPAD), lambda i: (0, 0)),
        ],
        out_specs=[
            pl.BlockSpec((br, F), lambda i: (i, 0)),
            pl.BlockSpec((br, NBPAD), lambda i: (i, 0)),
        ],
        out_shape=[
            jax.ShapeDtypeStruct((N, F), jnp.bfloat16),
            jax.ShapeDtypeStruct((N, NBPAD), jnp.float32),
        ],
        compiler_params=pltpu.CompilerParams(
            dimension_semantics=("arbitrary",),
        ),
    )(adj, y1, b1, bdiag)


def _l2_kernel(adj_ref, y_ref, b_ref, o_ref):
    acc = jnp.dot(adj_ref[...].astype(jnp.bfloat16), y_ref[...],
                  preferred_element_type=jnp.float32)
    o_ref[...] = acc + b_ref[...]


def _layer2_dense(adj, y2, b2, rows):
    br = 400
    return pl.pallas_call(
        _l2_kernel,
        grid=(rows // br,),
        in_specs=[
            pl.BlockSpec((br, N), lambda i: (i, 0)),
            pl.BlockSpec((N, F), lambda i: (0, 0)),
            pl.BlockSpec((1, F), lambda i: (0, 0)),
        ],
        out_specs=pl.BlockSpec((br, F), lambda i: (i, 0)),
        out_shape=jax.ShapeDtypeStruct((rows, F), jnp.float32),
        compiler_params=pltpu.CompilerParams(
            dimension_semantics=("arbitrary",),
        ),
    )(adj, y2, b2)


def _sc_l2(adjv, bs, y2p, b2, *, split):
    mesh = plsc.VectorSubcoreMesh(core_axis_name="c", subcore_axis_name="s")
    rows_sc = N - split
    rw = rows_sc // NW

    @functools.partial(
        pl.kernel,
        mesh=mesh,
        out_type=jax.ShapeDtypeStruct((rows_sc, F), jnp.float32),
        compiler_params=pltpu.CompilerParams(needs_layout_passes=False, use_tc_tiling_on_sc=False),
        scratch_types=[
            pltpu.VMEM((NBPAD,), jnp.float32),
            pltpu.VMEM((BCAP + 16,), jnp.int32),
            pltpu.VMEM((BCAP, BW), jnp.float32),
            pltpu.VMEM((ECAP + 16,), jnp.int32),
            pltpu.VMEM((ECAP + 16,), jnp.float32),
            pltpu.VMEM((ECAP, F), jnp.bfloat16),
            pltpu.VMEM((F,), jnp.float32),
            pltpu.VMEM((F,), jnp.float32),
            pltpu.SemaphoreType.DMA,
            pltpu.SemaphoreType.DMA,
        ],
    )
    def sc_kernel(adjv_hbm, bs_hbm, y2_hbm, b2_hbm, out_hbm,
                  bs_row, band_ids, band_buf, ecols, evals, y2_buf,
                  out_row, b2_v, sem_b, sem_y):
        wid = lax.axis_index("s") * 2 + lax.axis_index("c")
        r0 = wid * rw
        pltpu.sync_copy(b2_hbm, b2_v)
        lanes = lax.iota(jnp.int32, 16)
        zero16 = jnp.zeros((16,), jnp.int32)
        fzero16 = jnp.zeros((16,), jnp.float32)
        for k in range(BCAP // 16):
            band_ids[pl.ds(k * 16, 16)] = zero16
        for k in range(ECAP // 16):
            ecols[pl.ds(k * 16, 16)] = zero16

        def zrow(e, _):
            for k in range(F // 32):
                y2_buf[e, pl.ds(k * 32, 32)] = jnp.zeros((32,), jnp.bfloat16)
            return 0

        lax.fori_loop(0, ECAP, zrow, 0)

        def row_body(i, carry):
            r = split + r0 + i
            pltpu.sync_copy(bs_hbm.at[r], bs_row)
            for k in range(ECAP // 16):
                evals[pl.ds(k * 16, 16)] = fzero16
            nb = jnp.int32(0)
            base = r * NBAND
            for k in range(NBPAD // 16):
                v = bs_row[pl.ds(k * 16, 16)]
                m = v > 0.0
                cnt = plsc.all_reduce_population_count(m)[0]
                pos = plsc.cumsum(jnp.where(m, 1, 0))
                idx = jnp.where(m, jnp.minimum(nb + pos - 1, BCAP - 1),
                                BCAP)
                plsc.store_scatter(band_ids, [idx], base + k * 16 + lanes)
                nb = nb + cnt
            nb = jnp.minimum(nb, BCAP)

            ngb = (nb + 7) // 8
            for g in range(BCAP // 8):
                if g < NGB_UNC:
                    pltpu.make_async_copy(
                        adjv_hbm.at[band_ids.at[pl.ds(g * 8, 8)]],
                        band_buf.at[pl.ds(g * 8, 8)], sem_b).start()
                else:
                    @pl.when(g < ngb)
                    def _():
                        pltpu.make_async_copy(
                            adjv_hbm.at[band_ids.at[pl.ds(g * 8, 8)]],
                            band_buf.at[pl.ds(g * 8, 8)], sem_b).start()
            for g in range(BCAP // 8):
                if g < NGB_UNC:
                    pltpu.make_async_copy(
                        adjv_hbm.at[band_ids.at[pl.ds(g * 8, 8)]],
                        band_buf.at[pl.ds(g * 8, 8)], sem_b).wait()
                else:
                    @pl.when(g < ngb)
                    def _():
                        pltpu.make_async_copy(
                            adjv_hbm.at[band_ids.at[pl.ds(g * 8, 8)]],
                            band_buf.at[pl.ds(g * 8, 8)], sem_b).wait()

            def band_step(j, ec, fixed):
                bid = plsc.load_gather(band_ids, [jnp.full((16,), j,
                                                           jnp.int32)])
                colbase = (bid - base) * BW
                for k in range(BW // 16):
                    v = band_buf[j, pl.ds(k * 16, 16)]
                    m = v != 0.0
                    if fixed:
                        m = m & (jnp.full((16,), j, jnp.int32) < nb)
                    cnt = plsc.all_reduce_population_count(m)[0]
                    pos = plsc.cumsum(jnp.where(m, 1, 0))
                    idx = jnp.where(m, jnp.minimum(ec + pos - 1, ECAP - 1),
                                    ECAP)
                    plsc.store_scatter(ecols, [idx],
                                       colbase + k * 16 + lanes)
                    plsc.store_scatter(evals, [idx], v)
                    ec = ec + cnt
                return jnp.minimum(ec, ECAP)

            ec = jnp.int32(0)
            for j in range(NB_FIX):
                ec = band_step(j, ec, True)
            ec = lax.fori_loop(NB_FIX, nb,
                               lambda j, c: band_step(j, c, False), ec)

            nge = (ec + 7) // 8
            for g in range(ECAP // 8):
                if g < NGE_UNC:
                    pltpu.make_async_copy(
                        y2_hbm.at[ecols.at[pl.ds(g * 8, 8)]],
                        y2_buf.at[pl.ds(g * 8, 8)], sem_y).start()
                else:
                    @pl.when(g < nge)
                    def _():
                        pltpu.make_async_copy(
                            y2_hbm.at[ecols.at[pl.ds(g * 8, 8)]],
                            y2_buf.at[pl.ds(g * 8, 8)], sem_y).start()
            for g in range(ECAP // 8):
                if g < NGE_UNC:
                    pltpu.make_async_copy(
                        y2_hbm.at[ecols.at[pl.ds(g * 8, 8)]],
                        y2_buf.at[pl.ds(g * 8, 8)], sem_y).wait()
                else:
                    @pl.when(g < nge)
                    def _():
                        pltpu.make_async_copy(
                            y2_hbm.at[ecols.at[pl.ds(g * 8, 8)]],
                            y2_buf.at[pl.ds(g * 8, 8)], sem_y).wait()

            def fma_body(e, acc):
                vv = plsc.load_gather(evals, [jnp.full((16,), e, jnp.int32)])
                new = []
                for k in range(F // 32):
                    v32 = y2_buf[e, pl.ds(k * 32, 32)]
                    vi = plsc.bitcast(v32, jnp.int32)
                    lo = plsc.bitcast(vi << 16, jnp.float32)
                    hi = plsc.bitcast(vi & jnp.int32(-65536), jnp.float32)
                    new.append(acc[2 * k] + vv * lo)
                    new.append(acc[2 * k + 1] + vv * hi)
                return tuple(new)

            acc0 = tuple(jnp.zeros((16,), jnp.float32)
                         for _ in range(F // 16))
            acc = acc0
            for e in range(NE_FIX):
                acc = fma_body(e, acc)
            acc = lax.fori_loop(NE_FIX, ec, fma_body, acc)
            for k in range(F // 16):
                out_row[pl.ds(k * 16, 16)] = (
                    acc[k] + b2_v[pl.ds(k * 16, 16)])
            pltpu.sync_copy(out_row, out_hbm.at[r0 + i])
            return 0

        lax.fori_loop(0, rw, row_body, 0)

    return sc_kernel(adjv, bs, y2p, b2)


def kernel(x, adj, W1, b1, W2, b2):
    bdiag = (jnp.arange(N, dtype=jnp.int32)[:, None] // BW
             == jnp.arange(NBPAD, dtype=jnp.int32)[None, :]).astype(jnp.bfloat16)
    y1 = _feat_mm(x, W1)
    hr, bs = _layer1(adj, y1, b1.reshape(1, F), bdiag)
    y2 = _feat_mm(hr, W2)
    y2p = y2.reshape(N, F // 32, 2, 16).transpose(0, 1, 3, 2).reshape(N, F)
    adjv = adj.reshape(N * NBAND, BW)
    out_tc = _layer2_dense(adj, y2, b2.reshape(1, F), SPLIT)
    out_sc = _sc_l2(adjv, bs, y2p, b2, split=SPLIT)
    return jnp.concatenate([out_tc, out_sc], axis=0)


# speedup vs baseline: 11.4044x; 11.4044x over previous
import functools

import jax
import jax.numpy as jnp
from jax.experimental import pallas as pl
from jax.experimental.pallas import tpu as pltpu

N = 10000
F = 256


def _mm_kernel(x_ref, w_ref, o_ref):
    o_ref[...] = jnp.dot(x_ref[...], w_ref[...],
                         preferred_element_type=jnp.float32)


def _feat_mm(x, w):
    br = 1000
    return pl.pallas_call(
        _mm_kernel,
        grid=(N // br,),
        in_specs=[
            pl.BlockSpec((br, F), lambda i: (i, 0)),
            pl.BlockSpec((F, F), lambda i: (0, 0)),
        ],
        out_specs=pl.BlockSpec((br, F), lambda i: (i, 0)),
        out_shape=jax.ShapeDtypeStruct((N, F), jnp.float32),
    )(x, w)


def _agg_kernel(adj_ref, y_ref, b_ref, o_ref, *, relu):
    acc = jnp.dot(adj_ref[...], y_ref[...],
                  preferred_element_type=jnp.float32)
    acc = acc + b_ref[...]
    if relu:
        acc = jnp.maximum(acc, 0.0)
    o_ref[...] = acc


def _aggregate(adj, y, b, relu):
    br = 400
    return pl.pallas_call(
        functools.partial(_agg_kernel, relu=relu),
        grid=(N // br,),
        in_specs=[
            pl.BlockSpec((br, N), lambda i: (i, 0)),
            pl.BlockSpec((N, F), lambda i: (0, 0)),
            pl.BlockSpec((1, F), lambda i: (0, 0)),
        ],
        out_specs=pl.BlockSpec((br, F), lambda i: (i, 0)),
        out_shape=jax.ShapeDtypeStruct((N, F), jnp.float32),
        compiler_params=pltpu.CompilerParams(
            dimension_semantics=("arbitrary",),
        ),
    )(adj, y, b)


def kernel(x, adj, W1, b1, W2, b2):
    h = _aggregate(adj, _feat_mm(x, W1), b1.reshape(1, F), relu=True)
    out = _aggregate(adj, _feat_mm(h, W2), b2.reshape(1, F), relu=False)
    return out


# speedup vs baseline: 12.4616x; 1.0927x over previous
import functools

import jax
import jax.numpy as jnp
from jax.experimental import pallas as pl
from jax.experimental.pallas import tpu as pltpu

N = 10000
F = 256


def _layer_kernel(adj_ref, x_ref, w_ref, b_ref, o_ref, *, relu):
    agg = jnp.dot(adj_ref[...], x_ref[...],
                  preferred_element_type=jnp.float32)
    acc = jnp.dot(agg, w_ref[...], preferred_element_type=jnp.float32)
    acc = acc + b_ref[...]
    if relu:
        acc = jnp.maximum(acc, 0.0)
    o_ref[...] = acc


def _layer(adj, x, w, b, relu):
    br = 400
    return pl.pallas_call(
        functools.partial(_layer_kernel, relu=relu),
        grid=(N // br,),
        in_specs=[
            pl.BlockSpec((br, N), lambda i: (i, 0)),
            pl.BlockSpec((N, F), lambda i: (0, 0)),
            pl.BlockSpec((F, F), lambda i: (0, 0)),
            pl.BlockSpec((1, F), lambda i: (0, 0)),
        ],
        out_specs=pl.BlockSpec((br, F), lambda i: (i, 0)),
        out_shape=jax.ShapeDtypeStruct((N, F), jnp.float32),
        compiler_params=pltpu.CompilerParams(
            dimension_semantics=("arbitrary",),
        ),
    )(adj, x, w, b)


def kernel(x, adj, W1, b1, W2, b2):
    h = _layer(adj, x, W1, b1.reshape(1, F), relu=True)
    return _layer(adj, h, W2, b2.reshape(1, F), relu=False)
